# baseline (speedup 1.0000x reference)
import functools

import jax
import jax.numpy as jnp
from jax import lax
from jax.experimental import pallas as pl
from jax.experimental.pallas import tpu as pltpu
from jax.experimental.pallas import tpu_sc as plsc

EMBED = 768
MAX_POS = 64
LANES = 16
CHUNK = 32


@functools.cache
def _make_kernel(n_rows):
    info = plsc.get_sparse_core_info()
    nc, ns = info.num_cores, info.num_subcores
    nw = nc * ns
    rows_per_w = n_rows // nw
    n_chunks = rows_per_w // CHUNK
    n_pairs = n_chunks // 2
    mesh = plsc.VectorSubcoreMesh(core_axis_name="c", subcore_axis_name="s")

    @functools.partial(
        pl.kernel,
        out_type=jax.ShapeDtypeStruct((n_rows, EMBED), jnp.float32),
        mesh=mesh,
        scratch_types=[
            pltpu.VMEM((CHUNK,), jnp.int32),
            pltpu.VMEM((CHUNK,), jnp.int32),
            pltpu.VMEM((CHUNK, EMBED), jnp.float32),
            pltpu.VMEM((CHUNK, EMBED), jnp.float32),
            pltpu.VMEM((MAX_POS, EMBED), jnp.float32),
            pltpu.VMEM((CHUNK,), jnp.int32),
            pltpu.SemaphoreType.DMA,
            pltpu.SemaphoreType.DMA,
        ],
    )
    def k(ids_hbm, table_hbm, pos_hbm, out_hbm,
          idx0_v, idx1_v, rows0_v, rows1_v, pos_v, iota_v, sem0, sem1):
        wid = lax.axis_index("s") * nc + lax.axis_index("c")
        base = wid * rows_per_w
        pltpu.sync_copy(pos_hbm, pos_v)

        def issue(c, idx_v, rows_v, sem):
            pltpu.sync_copy(ids_hbm.at[pl.ds(c, CHUNK)], idx_v)
            pltpu.async_copy(table_hbm.at[idx_v], rows_v, sem)

        def finish(c, idx_v, rows_v, sem, pos_off):
            pltpu.make_async_copy(table_hbm.at[idx_v], rows_v, sem).wait()

            def add_row(r, carry):
                for d in range(EMBED // LANES):
                    col = d * LANES
                    rows_v[r, pl.ds(col, LANES)] = (
                        rows_v[r, pl.ds(col, LANES)]
                        + pos_v[pos_off + r, pl.ds(col, LANES)]
                    )
                return carry

            lax.fori_loop(0, CHUNK, add_row, 0)

            pltpu.sync_copy(rows_v, out_hbm.at[pl.ds(c, CHUNK)])

        issue(base, idx0_v, rows0_v, sem0)

        def pair_body(i, carry):
            c0 = base + (2 * i) * CHUNK
            c1 = c0 + CHUNK
            issue(c1, idx1_v, rows1_v, sem1)
            finish(c0, idx0_v, rows0_v, sem0, 0)

            @pl.when(i + 1 < n_pairs)
            def _():
                issue(c1 + CHUNK, idx0_v, rows0_v, sem0)

            finish(c1, idx1_v, rows1_v, sem1, CHUNK)
            return carry

        lax.fori_loop(0, n_pairs, pair_body, 0)

    return k


def kernel(input_ids, token_embedding, position_embedding):
    b, s = input_ids.shape
    ids_flat = input_ids.reshape(b * s).astype(jnp.int32)
    out = _make_kernel(b * s)(ids_flat, token_embedding, position_embedding)
    return out.reshape(b, s, EMBED)


# speedup vs baseline: 1.2849x; 1.2849x over previous
import functools

import jax
import jax.numpy as jnp
from jax import lax
from jax.experimental import pallas as pl
from jax.experimental.pallas import tpu as pltpu
from jax.experimental.pallas import tpu_sc as plsc

EMBED = 768
MAX_POS = 64
LANES = 16
CHUNK = 32


@functools.cache
def _make_kernel(n_rows):
    info = plsc.get_sparse_core_info()
    nc, ns = info.num_cores, info.num_subcores
    nw = nc * ns
    rows_per_w = n_rows // nw
    n_chunks = rows_per_w // CHUNK
    n_pairs = n_chunks // 2
    mesh = plsc.VectorSubcoreMesh(core_axis_name="c", subcore_axis_name="s")

    @functools.partial(
        pl.kernel,
        out_type=jax.ShapeDtypeStruct((n_rows, EMBED), jnp.float32),
        mesh=mesh,
        scratch_types=[
            pltpu.VMEM((CHUNK,), jnp.int32),
            pltpu.VMEM((CHUNK,), jnp.int32),
            pltpu.VMEM((CHUNK, EMBED), jnp.float32),
            pltpu.VMEM((CHUNK, EMBED), jnp.float32),
            pltpu.VMEM((MAX_POS, EMBED), jnp.float32),
            pltpu.SemaphoreType.DMA,
            pltpu.SemaphoreType.DMA,
            pltpu.SemaphoreType.DMA,
            pltpu.SemaphoreType.DMA,
        ],
    )
    def k(ids_hbm, table_hbm, pos_hbm, out_hbm,
          idx0_v, idx1_v, rows0_v, rows1_v, pos_v,
          semg0, semg1, semi0, semi1):
        wid = lax.axis_index("s") * nc + lax.axis_index("c")
        base = wid * rows_per_w
        cbase = wid * n_chunks
        pltpu.sync_copy(pos_hbm, pos_v)

        def id_load(c, idx_v, semi):
            return pltpu.make_async_copy(ids_hbm.at[cbase + c], idx_v, semi)

        def gather(idx_v, rows_v, semg):
            return pltpu.make_async_copy(table_hbm.at[idx_v], rows_v, semg)

        def addpos(rows_v, pos_off):
            def row(r, carry):
                for d in range(EMBED // LANES):
                    sl = pl.ds(d * LANES, LANES)
                    plsc.addupdate(rows_v.at[r, sl], pos_v[pos_off + r, sl])
                return carry

            lax.fori_loop(0, CHUNK, row, 0)

        id_load(0, idx0_v, semi0).start()
        id_load(0, idx0_v, semi0).wait()
        gather(idx0_v, rows0_v, semg0).start()
        id_load(1, idx1_v, semi1).start()

        def pair_body(i, carry):
            c0 = 2 * i
            c1 = c0 + 1
            id_load(c1, idx1_v, semi1).wait()
            gather(idx1_v, rows1_v, semg1).start()

            gather(idx0_v, rows0_v, semg0).wait()

            @pl.when(i + 1 < n_pairs)
            def _():
                id_load(c0 + 2, idx0_v, semi0).start()

            addpos(rows0_v, 0)
            pltpu.sync_copy(rows0_v, out_hbm.at[pl.ds(base + c0 * CHUNK, CHUNK)])

            @pl.when(i + 1 < n_pairs)
            def _():
                id_load(c0 + 2, idx0_v, semi0).wait()
                gather(idx0_v, rows0_v, semg0).start()

            gather(idx1_v, rows1_v, semg1).wait()

            @pl.when(i + 1 < n_pairs)
            def _():
                id_load(c1 + 2, idx1_v, semi1).start()

            addpos(rows1_v, CHUNK)
            pltpu.sync_copy(rows1_v, out_hbm.at[pl.ds(base + c1 * CHUNK, CHUNK)])
            return carry

        lax.fori_loop(0, n_pairs, pair_body, 0)

    return k


def kernel(input_ids, token_embedding, position_embedding):
    b, s = input_ids.shape
    n_rows = b * s
    ids2 = input_ids.reshape(n_rows // CHUNK, CHUNK).astype(jnp.int32)
    out = _make_kernel(n_rows)(ids2, token_embedding, position_embedding)
    return out.reshape(b, s, EMBED)
